# baseline (speedup 1.0000x reference)
import functools

import jax
import jax.numpy as jnp
from jax import lax
from jax.experimental import pallas as pl
from jax.experimental.pallas import tpu as pltpu
from jax.experimental.pallas import tpu_sc as plsc

NN = 10000
EE = 320000
DD = 128
HEADS1 = 8
BB = 4096

LL = 16
_SC_PARAMS = pltpu.CompilerParams(needs_layout_passes=False,
                                  use_tc_tiling_on_sc=False)
NCORE, NSUB = 2, 16
NWORK = NCORE * NSUB
GROUP = 32
NGROUPS = 324
NWIN = 6
WGROUPS = NGROUPS // NWIN
EPAD = NWORK * NGROUPS * GROUP
NPAD = 10240
ROWS_PER_TILE = NPAD // NSUB


def _f32(x):
    return x.astype(jnp.float32)


def _perm(v, idx):
    return v.at[idx].get(mode="promise_in_bounds")


def _br3(h):
    return ((h & 1) << 2) | (h & 2) | ((h & 4) >> 2)


def _mm2(xp, Wl, Wr):
    R = 512

    def body(x_ref, wl_ref, wr_ref, xl_ref, xr_ref):
        x = x_ref[...]
        dn = (((1,), (1,)), ((), ()))
        xl_ref[...] = lax.dot_general(x, wl_ref[...], dn,
                                      preferred_element_type=jnp.float32)
        xr_ref[...] = lax.dot_general(x, wr_ref[...], dn,
                                      preferred_element_type=jnp.float32)

    return pl.pallas_call(
        body,
        grid=(NPAD // R,),
        in_specs=[
            pl.BlockSpec((R, DD), lambda i: (i, 0)),
            pl.BlockSpec((DD, DD), lambda i: (0, 0)),
            pl.BlockSpec((DD, DD), lambda i: (0, 0)),
        ],
        out_specs=[pl.BlockSpec((R, DD), lambda i: (i, 0))] * 2,
        out_shape=[jax.ShapeDtypeStruct((NPAD, DD), jnp.float32)] * 2,
    )(xp, Wl, Wr)


def _combine1(u, den, b1r, Wl2, Wr2):
    R = 512

    def body(u_ref, d_ref, b_ref, wl_ref, wr_ref, h_ref, xl_ref, xr_ref):
        us = u_ref[0] + u_ref[1]
        d = d_ref[0] + d_ref[1]
        d8 = d[:, :HEADS1] + 1e-16
        db = jnp.reshape(
            jnp.broadcast_to(d8[:, :, None], (R, HEADS1, DD // HEADS1)),
            (R, DD))
        h = jnp.maximum(us / db + b_ref[...], 0.0)
        h_ref[...] = h
        dn = (((1,), (1,)), ((), ()))
        xl_ref[...] = lax.dot_general(h, wl_ref[...], dn,
                                      preferred_element_type=jnp.float32)
        xr_ref[...] = lax.dot_general(h, wr_ref[...], dn,
                                      preferred_element_type=jnp.float32)

    return pl.pallas_call(
        body,
        grid=(NPAD // R,),
        in_specs=[
            pl.BlockSpec((2, R, DD), lambda i: (0, i, 0)),
            pl.BlockSpec((2, R, LL), lambda i: (0, i, 0)),
            pl.BlockSpec((1, DD), lambda i: (0, 0)),
            pl.BlockSpec((DD, DD), lambda i: (0, 0)),
            pl.BlockSpec((DD, DD), lambda i: (0, 0)),
        ],
        out_specs=[pl.BlockSpec((R, DD), lambda i: (i, 0))] * 3,
        out_shape=[jax.ShapeDtypeStruct((NPAD, DD), jnp.float32)] * 3,
    )(u, den, b1r, Wl2, Wr2)


def _combine2(u, den, b2r):
    R = 512

    def body(u_ref, d_ref, b_ref, h_ref):
        us = u_ref[0] + u_ref[1]
        d = d_ref[0] + d_ref[1]
        d1 = d[:, :1] + 1e-16
        h_ref[...] = us / jnp.broadcast_to(d1, (R, DD)) + b_ref[...]

    return pl.pallas_call(
        body,
        grid=(NPAD // R,),
        in_specs=[
            pl.BlockSpec((2, R, DD), lambda i: (0, i, 0)),
            pl.BlockSpec((2, R, LL), lambda i: (0, i, 0)),
            pl.BlockSpec((1, DD), lambda i: (0, 0)),
        ],
        out_specs=pl.BlockSpec((R, DD), lambda i: (i, 0)),
        out_shape=jax.ShapeDtypeStruct((NPAD, DD), jnp.float32),
    )(u, den, b2r)


def _edge_pass(nheads):
    chead = DD // nheads
    mesh = plsc.VectorSubcoreMesh(core_axis_name="c", subcore_axis_name="s")
    out_type = [
        jax.ShapeDtypeStruct((NCORE, NPAD, DD), jnp.float32),
        jax.ShapeDtypeStruct((NCORE, NPAD, LL), jnp.float32),
    ]
    DW = DD + LL
    scratch = [
        pltpu.VMEM((WGROUPS, GROUP), jnp.int32),
        pltpu.VMEM((WGROUPS, GROUP), jnp.int32),
        pltpu.VMEM((2, GROUP, DD), jnp.float32),
        pltpu.VMEM((2, GROUP, DD), jnp.float32),
        pltpu.VMEM((2, GROUP, DW), jnp.float32),
        pltpu.VMEM((GROUP, LL), jnp.float32),
        pltpu.VMEM((DD,), jnp.float32),
        pltpu.VMEM((LL, DW), jnp.float32),
        pltpu.VMEM_SHARED((NPAD, DW), jnp.float32),
        pltpu.SemaphoreType.DMA,
        pltpu.SemaphoreType.DMA,
        pltpu.SemaphoreType.DMA,
        pltpu.SemaphoreType.DMA,
        pltpu.SemaphoreType.DMA,
        pltpu.SemaphoreType.DMA,
    ]

    @functools.partial(pl.kernel, out_type=out_type, mesh=mesh,
                       scratch_types=scratch, compiler_params=_SC_PARAMS)
    def k(xl_hbm, xr_hbm, src_hbm, dst_hbm, att_hbm, out_hbm, den_hbm,
          swin, dwin, xbuf, rbuf, rowbuf, dstage, att_v, zbuf,
          oacc, sgx0, sgr0, sgx1, sgr1, sso0, sso1):
        cid = lax.axis_index("c")
        sid = lax.axis_index("s")
        wid = sid * NCORE + cid

        pltpu.sync_copy(att_hbm, att_v)

        zv = jnp.zeros((LL,), jnp.float32)
        for r in range(LL):
            for j in range(DW // LL):
                zbuf[r, pl.ds(j * LL, LL)] = zv

        r0 = sid * ROWS_PER_TILE

        def zbody(j, carry):
            pltpu.sync_copy(zbuf, oacc.at[pl.ds(r0 + j * LL, LL), :])
            return carry

        lax.fori_loop(0, ROWS_PER_TILE // LL, zbody, 0)
        plsc.subcore_barrier()

        iot = lax.iota(jnp.int32, LL)

        def wloop(w, carry):
            pltpu.sync_copy(src_hbm.at[wid, pl.ds(w * WGROUPS, WGROUPS), :],
                            swin)
            pltpu.sync_copy(dst_hbm.at[wid, pl.ds(w * WGROUPS, WGROUPS), :],
                            dwin)

            x8 = iot ^ 8
            x4 = iot ^ 4
            x2 = iot ^ 2
            x1 = iot ^ 1
            m8 = iot < 8
            m4 = (iot & 7) < 4
            m2 = (iot & 3) < 2
            invmap = (((iot & 1) << 3) | ((iot & 2) << 1) | ((iot & 4) >> 1))
            attv = [att_v[pl.ds(b * LL, LL)] for b in range(DD // LL)]

            def compute(slot):
                def ebody(e, carry2):
                    xv = [xbuf[slot, e, pl.ds(b * LL, LL)]
                          for b in range(DD // LL)]
                    w = []
                    for b in range(DD // LL):
                        z = xv[b] + rbuf[slot, e, pl.ds(b * LL, LL)]
                        t = jnp.maximum(z, 0.2 * z)
                        w.append(t * attv[b])
                    if nheads == 8:
                        u = [v + _perm(v, x8) for v in w]
                        z4 = [jnp.where(m8, u[2 * i], _perm(u[2 * i + 1], x8))
                              for i in range(4)]
                        z4 = [v + _perm(v, x4) for v in z4]
                        y2 = [jnp.where(m4, z4[2 * i],
                                        _perm(z4[2 * i + 1], x4))
                              for i in range(2)]
                        y2 = [v + _perm(v, x2) for v in y2]
                        p = jnp.where(m2, y2[0], _perm(y2[1], x2))
                        s = p + _perm(p, x1)
                        ex = jnp.exp(s)
                        rowbuf[slot, e, pl.ds(DD, LL)] = _perm(ex, invmap)
                        for b in range(DD // LL):
                            exb = _perm(ex, jnp.full((LL,), 2 * _br3(b),
                                                     jnp.int32))
                            rowbuf[slot, e, pl.ds(b * LL, LL)] = xv[b] * exb
                    else:
                        s = w[0]
                        for b in range(1, DD // LL):
                            s = s + w[b]
                        s = s + _perm(s, x8)
                        s = s + _perm(s, x4)
                        s = s + _perm(s, x2)
                        s = s + _perm(s, x1)
                        ex = jnp.exp(s)
                        rowbuf[slot, e, pl.ds(DD, LL)] = ex
                        for b in range(DD // LL):
                            rowbuf[slot, e, pl.ds(b * LL, LL)] = xv[b] * ex
                    return carry2

                lax.fori_loop(0, GROUP, ebody, 0)

            def pbody(p, carry1):
                g0 = 2 * p
                g1 = g0 + 1
                cx0 = pltpu.async_copy(xl_hbm.at[swin.at[g0]], xbuf.at[0],
                                       sgx0)
                cr0 = pltpu.async_copy(xr_hbm.at[dwin.at[g0]], rbuf.at[0],
                                       sgr0)
                cx1 = pltpu.async_copy(xl_hbm.at[swin.at[g1]], xbuf.at[1],
                                       sgx1)
                cr1 = pltpu.async_copy(xr_hbm.at[dwin.at[g1]], rbuf.at[1],
                                       sgr1)
                cx0.wait()
                cr0.wait()
                compute(0)
                co0 = pltpu.async_copy(rowbuf.at[0], oacc.at[dwin.at[g0]],
                                       sso0, add=True)
                cx1.wait()
                cr1.wait()
                compute(1)
                co1 = pltpu.async_copy(rowbuf.at[1], oacc.at[dwin.at[g1]],
                                       sso1, add=True)
                co0.wait()
                co1.wait()
                return carry1

            lax.fori_loop(0, WGROUPS // 2, pbody, 0)
            return carry

        lax.fori_loop(0, NWIN, wloop, 0)
        plsc.subcore_barrier()

        def wbody(j, carry):
            rr = r0 + j * GROUP
            pltpu.sync_copy(oacc.at[pl.ds(rr, GROUP), pl.ds(0, DD)],
                            xbuf.at[0])
            pltpu.sync_copy(xbuf.at[0], out_hbm.at[cid, pl.ds(rr, GROUP), :])
            pltpu.sync_copy(oacc.at[pl.ds(rr, GROUP), pl.ds(DD, LL)], dstage)
            pltpu.sync_copy(dstage, den_hbm.at[cid, pl.ds(rr, GROUP), :])
            return carry

        lax.fori_loop(0, ROWS_PER_TILE // GROUP, wbody, 0)

    return k


BGROUPS = BB // NWORK // GROUP


def _cls_kernel():
    mesh = plsc.VectorSubcoreMesh(core_axis_name="c", subcore_axis_name="s")
    out_type = jax.ShapeDtypeStruct((BB,), jnp.float32)
    scratch = [
        pltpu.VMEM((BGROUPS, GROUP), jnp.int32),
        pltpu.VMEM((BGROUPS, GROUP), jnp.int32),
        pltpu.VMEM((GROUP, DD), jnp.float32),
        pltpu.VMEM((GROUP, DD), jnp.float32),
        pltpu.VMEM((GROUP, DD), jnp.float32),
        pltpu.VMEM((GROUP, DD), jnp.float32),
        pltpu.VMEM((4 * DD,), jnp.float32),
        pltpu.VMEM((LL,), jnp.float32),
        pltpu.VMEM((GROUP,), jnp.float32),
        pltpu.SemaphoreType.DMA,
    ]

    @functools.partial(pl.kernel, out_type=out_type, mesh=mesh,
                       scratch_types=scratch, compiler_params=_SC_PARAMS)
    def k(van_hbm, h_hbm, a1_hbm, a2_hbm, wc_hbm, bc_hbm, out_hbm,
          a1_v, a2_v, vb1, vb2, hb1, hb2, wc_v, bc_v, lbuf, sem):
        cid = lax.axis_index("c")
        sid = lax.axis_index("s")
        wid = sid * NCORE + cid
        pltpu.sync_copy(a1_hbm.at[wid], a1_v)
        pltpu.sync_copy(a2_hbm.at[wid], a2_v)
        pltpu.sync_copy(wc_hbm, wc_v)
        pltpu.sync_copy(bc_hbm, bc_v)
        iot = lax.iota(jnp.int32, LL)

        def gbody(g, carry):
            c1 = pltpu.async_copy(van_hbm.at[a1_v.at[g]], vb1, sem)
            c2 = pltpu.async_copy(van_hbm.at[a2_v.at[g]], vb2, sem)
            c3 = pltpu.async_copy(h_hbm.at[a1_v.at[g]], hb1, sem)
            c4 = pltpu.async_copy(h_hbm.at[a2_v.at[g]], hb2, sem)
            c1.wait()
            c2.wait()
            c3.wait()
            c4.wait()

            def sbody(k4, carry2):
                rowi = iot + k4 * LL
                acc = jnp.zeros((LL,), jnp.float32)
                for blk in range(DD // LL):
                    w0 = wc_v[pl.ds(blk * LL, LL)]
                    w1 = wc_v[pl.ds(DD + blk * LL, LL)]
                    w2 = wc_v[pl.ds(2 * DD + blk * LL, LL)]
                    w3 = wc_v[pl.ds(3 * DD + blk * LL, LL)]
                    for j in range(LL):
                        c = blk * LL + j
                        colv = jnp.full((LL,), c, jnp.int32)
                        acc = acc + plsc.load_gather(vb1, [rowi, colv]) * w0[j]
                        acc = acc + plsc.load_gather(vb2, [rowi, colv]) * w1[j]
                        acc = acc + plsc.load_gather(hb1, [rowi, colv]) * w2[j]
                        acc = acc + plsc.load_gather(hb2, [rowi, colv]) * w3[j]
                acc = acc + bc_v[pl.ds(0, LL)][0]
                lbuf[pl.ds(k4 * LL, LL)] = acc
                return carry2

            lax.fori_loop(0, GROUP // LL, sbody, 0)
            pltpu.sync_copy(
                lbuf, out_hbm.at[pl.ds(wid * (BGROUPS * GROUP) + g * GROUP,
                                       GROUP)])
            return carry

        lax.fori_loop(0, BGROUPS, gbody, 0)

    return k


_EDGE8 = _edge_pass(8)
_EDGE1 = _edge_pass(1)
_CLS = _cls_kernel()


def kernel(gnn_x, vanilla, edge_index, article1_idx, article2_idx,
           Wl1, Wr1, att1, b1, Wl2, Wr2, att2, b2, Wc, bc):
    loops = jnp.arange(NN, dtype=edge_index.dtype)
    src = jnp.concatenate([edge_index[0], loops])
    dst = jnp.concatenate([edge_index[1], loops])
    pad = EPAD - src.shape[0]
    padv = jnp.full((pad,), NN, dtype=jnp.int32)
    srcp = jnp.concatenate([src, padv]).reshape(NWORK, NGROUPS, GROUP)
    dstp = jnp.concatenate([dst, padv]).reshape(NWORK, NGROUPS, GROUP)
    xp = jnp.pad(_f32(gnn_x), ((0, NPAD - NN), (0, 0)))

    xl1, xr1 = _mm2(xp, _f32(Wl1), _f32(Wr1))
    u1, d1 = _EDGE8(xl1, xr1, srcp, dstp, _f32(att1).reshape(-1))
    h1, xl2, xr2 = _combine1(u1, d1, _f32(b1).reshape(1, DD),
                             _f32(Wl2), _f32(Wr2))

    u2, d2 = _EDGE1(xl2, xr2, srcp, dstp, _f32(att2).reshape(-1))
    h2 = _combine2(u2, d2, _f32(b2).reshape(1, DD))

    a1p = article1_idx.reshape(NWORK, BGROUPS, GROUP)
    a2p = article2_idx.reshape(NWORK, BGROUPS, GROUP)
    logits = _CLS(_f32(vanilla), h2, a1p, a2p,
                  _f32(Wc).reshape(-1), jnp.pad(_f32(bc), (0, LL - 1)))
    return logits.reshape(BB, 1)


# speedup vs baseline: 1.0742x; 1.0742x over previous
import functools

import jax
import jax.numpy as jnp
from jax import lax
from jax.experimental import pallas as pl
from jax.experimental.pallas import tpu as pltpu
from jax.experimental.pallas import tpu_sc as plsc

NN = 10000
EE = 320000
DD = 128
HEADS1 = 8
BB = 4096

LL = 16
_SC_PARAMS = pltpu.CompilerParams(needs_layout_passes=False,
                                  use_tc_tiling_on_sc=False)
NCORE, NSUB = 2, 16
NWORK = NCORE * NSUB
GROUP = 32
NGROUPS = 324
NWIN = 6
WGROUPS = NGROUPS // NWIN
EPAD = NWORK * NGROUPS * GROUP
NPAD = 10240
ROWS_PER_TILE = NPAD // NSUB


def _f32(x):
    return x.astype(jnp.float32)


def _perm(v, idx):
    return v.at[idx].get(mode="promise_in_bounds")


def _br3(h):
    return ((h & 1) << 2) | (h & 2) | ((h & 4) >> 2)


def _mm2(xp, Wl, Wr):
    R = 512

    def body(x_ref, wl_ref, wr_ref, xl_ref, xr_ref):
        x = x_ref[...]
        dn = (((1,), (1,)), ((), ()))
        xl_ref[...] = lax.dot_general(x, wl_ref[...], dn,
                                      preferred_element_type=jnp.float32)
        xr_ref[...] = lax.dot_general(x, wr_ref[...], dn,
                                      preferred_element_type=jnp.float32)

    return pl.pallas_call(
        body,
        grid=(NPAD // R,),
        in_specs=[
            pl.BlockSpec((R, DD), lambda i: (i, 0)),
            pl.BlockSpec((DD, DD), lambda i: (0, 0)),
            pl.BlockSpec((DD, DD), lambda i: (0, 0)),
        ],
        out_specs=[pl.BlockSpec((R, DD), lambda i: (i, 0))] * 2,
        out_shape=[jax.ShapeDtypeStruct((NPAD, DD), jnp.float32)] * 2,
    )(xp, Wl, Wr)


def _combine1(u, den, b1r, Wl2, Wr2):
    R = 512

    def body(u_ref, d_ref, b_ref, wl_ref, wr_ref, h_ref, xl_ref, xr_ref):
        us = u_ref[0] + u_ref[1]
        d = d_ref[0] + d_ref[1]
        d8 = d[:, :HEADS1] + 1e-16
        db = jnp.reshape(
            jnp.broadcast_to(d8[:, :, None], (R, HEADS1, DD // HEADS1)),
            (R, DD))
        h = jnp.maximum(us / db + b_ref[...], 0.0)
        h_ref[...] = h
        dn = (((1,), (1,)), ((), ()))
        xl_ref[...] = lax.dot_general(h, wl_ref[...], dn,
                                      preferred_element_type=jnp.float32)
        xr_ref[...] = lax.dot_general(h, wr_ref[...], dn,
                                      preferred_element_type=jnp.float32)

    return pl.pallas_call(
        body,
        grid=(NPAD // R,),
        in_specs=[
            pl.BlockSpec((2, R, DD), lambda i: (0, i, 0)),
            pl.BlockSpec((2, R, LL), lambda i: (0, i, 0)),
            pl.BlockSpec((1, DD), lambda i: (0, 0)),
            pl.BlockSpec((DD, DD), lambda i: (0, 0)),
            pl.BlockSpec((DD, DD), lambda i: (0, 0)),
        ],
        out_specs=[pl.BlockSpec((R, DD), lambda i: (i, 0))] * 3,
        out_shape=[jax.ShapeDtypeStruct((NPAD, DD), jnp.float32)] * 3,
    )(u, den, b1r, Wl2, Wr2)


def _combine2(u, den, b2r):
    R = 512

    def body(u_ref, d_ref, b_ref, h_ref):
        us = u_ref[0] + u_ref[1]
        d = d_ref[0] + d_ref[1]
        d1 = d[:, :1] + 1e-16
        h_ref[...] = us / jnp.broadcast_to(d1, (R, DD)) + b_ref[...]

    return pl.pallas_call(
        body,
        grid=(NPAD // R,),
        in_specs=[
            pl.BlockSpec((2, R, DD), lambda i: (0, i, 0)),
            pl.BlockSpec((2, R, LL), lambda i: (0, i, 0)),
            pl.BlockSpec((1, DD), lambda i: (0, 0)),
        ],
        out_specs=pl.BlockSpec((R, DD), lambda i: (i, 0)),
        out_shape=jax.ShapeDtypeStruct((NPAD, DD), jnp.float32),
    )(u, den, b2r)


def _edge_pass(nheads):
    chead = DD // nheads
    mesh = plsc.VectorSubcoreMesh(core_axis_name="c", subcore_axis_name="s")
    out_type = [
        jax.ShapeDtypeStruct((NCORE, NPAD, DD), jnp.float32),
        jax.ShapeDtypeStruct((NCORE, NPAD, LL), jnp.float32),
    ]
    scratch = [
        pltpu.VMEM((WGROUPS, GROUP), jnp.int32),
        pltpu.VMEM((WGROUPS, GROUP), jnp.int32),
        pltpu.VMEM((2, GROUP, DD), jnp.float32),
        pltpu.VMEM((2, GROUP, DD), jnp.float32),
        pltpu.VMEM((2, GROUP, DD), jnp.float32),
        pltpu.VMEM((2, GROUP, LL), jnp.float32),
        pltpu.VMEM((DD,), jnp.float32),
        pltpu.VMEM((LL, DD), jnp.float32),
        pltpu.VMEM_SHARED((NPAD, DD), jnp.float32),
        pltpu.VMEM_SHARED((NPAD, LL), jnp.float32),
        pltpu.SemaphoreType.DMA,
        pltpu.SemaphoreType.DMA,
        pltpu.SemaphoreType.DMA,
        pltpu.SemaphoreType.DMA,
        pltpu.SemaphoreType.DMA,
        pltpu.SemaphoreType.DMA,
        pltpu.SemaphoreType.DMA,
        pltpu.SemaphoreType.DMA,
    ]

    @functools.partial(pl.kernel, out_type=out_type, mesh=mesh,
                       scratch_types=scratch, compiler_params=_SC_PARAMS)
    def k(xl_hbm, xr_hbm, src_hbm, dst_hbm, att_hbm, out_hbm, den_hbm,
          swin, dwin, xbuf, rbuf, rowbuf, rowdbuf, att_v, zbuf,
          oacc, dacc, sgx0, sgr0, sgx1, sgr1, sso0, ssd0, sso1, ssd1):
        cid = lax.axis_index("c")
        sid = lax.axis_index("s")
        wid = sid * NCORE + cid

        pltpu.sync_copy(att_hbm, att_v)

        zv = jnp.zeros((LL,), jnp.float32)
        for r in range(LL):
            for j in range(DD // LL):
                zbuf[r, pl.ds(j * LL, LL)] = zv

        r0 = sid * ROWS_PER_TILE

        def zbody(j, carry):
            pltpu.sync_copy(zbuf, oacc.at[pl.ds(r0 + j * LL, LL), :])
            pltpu.sync_copy(zbuf.at[:, pl.ds(0, LL)],
                            dacc.at[pl.ds(r0 + j * LL, LL), :])
            return carry

        lax.fori_loop(0, ROWS_PER_TILE // LL, zbody, 0)
        plsc.subcore_barrier()

        iot = lax.iota(jnp.int32, LL)

        def wloop(w, carry):
            pltpu.sync_copy(src_hbm.at[wid, pl.ds(w * WGROUPS, WGROUPS), :],
                            swin)
            pltpu.sync_copy(dst_hbm.at[wid, pl.ds(w * WGROUPS, WGROUPS), :],
                            dwin)

            x8 = iot ^ 8
            x4 = iot ^ 4
            x2 = iot ^ 2
            x1 = iot ^ 1
            m8 = iot < 8
            m4 = (iot & 7) < 4
            m2 = (iot & 3) < 2
            invmap = (((iot & 1) << 3) | ((iot & 2) << 1) | ((iot & 4) >> 1))
            attv = [att_v[pl.ds(b * LL, LL)] for b in range(DD // LL)]

            def compute(slot):
                def ebody(e, carry2):
                    xv = [xbuf[slot, e, pl.ds(b * LL, LL)]
                          for b in range(DD // LL)]
                    w = []
                    for b in range(DD // LL):
                        z = xv[b] + rbuf[slot, e, pl.ds(b * LL, LL)]
                        t = jnp.maximum(z, 0.2 * z)
                        w.append(t * attv[b])
                    if nheads == 8:
                        u = [v + _perm(v, x8) for v in w]
                        z4 = [jnp.where(m8, u[2 * i], _perm(u[2 * i + 1], x8))
                              for i in range(4)]
                        z4 = [v + _perm(v, x4) for v in z4]
                        y2 = [jnp.where(m4, z4[2 * i],
                                        _perm(z4[2 * i + 1], x4))
                              for i in range(2)]
                        y2 = [v + _perm(v, x2) for v in y2]
                        p = jnp.where(m2, y2[0], _perm(y2[1], x2))
                        s = p + _perm(p, x1)
                        ex = jnp.exp(s)
                        rowdbuf[slot, e, :] = _perm(ex, invmap)
                        for b in range(DD // LL):
                            exb = _perm(ex, jnp.full((LL,), 2 * _br3(b),
                                                     jnp.int32))
                            rowbuf[slot, e, pl.ds(b * LL, LL)] = xv[b] * exb
                    else:
                        s = w[0]
                        for b in range(1, DD // LL):
                            s = s + w[b]
                        s = s + _perm(s, x8)
                        s = s + _perm(s, x4)
                        s = s + _perm(s, x2)
                        s = s + _perm(s, x1)
                        ex = jnp.exp(s)
                        rowdbuf[slot, e, :] = ex
                        for b in range(DD // LL):
                            rowbuf[slot, e, pl.ds(b * LL, LL)] = xv[b] * ex
                    return carry2

                lax.fori_loop(0, GROUP, ebody, 0)

            def pbody(p, carry1):
                g0 = 2 * p
                g1 = g0 + 1
                cx0 = pltpu.async_copy(xl_hbm.at[swin.at[g0]], xbuf.at[0],
                                       sgx0)
                cr0 = pltpu.async_copy(xr_hbm.at[dwin.at[g0]], rbuf.at[0],
                                       sgr0)
                cx1 = pltpu.async_copy(xl_hbm.at[swin.at[g1]], xbuf.at[1],
                                       sgx1)
                cr1 = pltpu.async_copy(xr_hbm.at[dwin.at[g1]], rbuf.at[1],
                                       sgr1)
                cx0.wait()
                cr0.wait()
                compute(0)
                co0 = pltpu.async_copy(rowbuf.at[0], oacc.at[dwin.at[g0]],
                                       sso0, add=True)
                cd0 = pltpu.async_copy(rowdbuf.at[0], dacc.at[dwin.at[g0]],
                                       ssd0, add=True)
                cx1.wait()
                cr1.wait()
                compute(1)
                co1 = pltpu.async_copy(rowbuf.at[1], oacc.at[dwin.at[g1]],
                                       sso1, add=True)
                cd1 = pltpu.async_copy(rowdbuf.at[1], dacc.at[dwin.at[g1]],
                                       ssd1, add=True)
                co0.wait()
                cd0.wait()
                co1.wait()
                cd1.wait()
                return carry1

            lax.fori_loop(0, WGROUPS // 2, pbody, 0)
            return carry

        lax.fori_loop(0, NWIN, wloop, 0)
        plsc.subcore_barrier()

        def wbody(j, carry):
            rr = r0 + j * GROUP
            pltpu.sync_copy(oacc.at[pl.ds(rr, GROUP), :], xbuf.at[0])
            pltpu.sync_copy(xbuf.at[0], out_hbm.at[cid, pl.ds(rr, GROUP), :])
            pltpu.sync_copy(dacc.at[pl.ds(rr, GROUP), :], rowdbuf.at[0])
            pltpu.sync_copy(rowdbuf.at[0],
                            den_hbm.at[cid, pl.ds(rr, GROUP), :])
            return carry

        lax.fori_loop(0, ROWS_PER_TILE // GROUP, wbody, 0)

    return k


BGROUPS = BB // NWORK // GROUP


def _cls_kernel():
    mesh = plsc.VectorSubcoreMesh(core_axis_name="c", subcore_axis_name="s")
    out_type = jax.ShapeDtypeStruct((BB,), jnp.float32)
    scratch = [
        pltpu.VMEM((BGROUPS, GROUP), jnp.int32),
        pltpu.VMEM((BGROUPS, GROUP), jnp.int32),
        pltpu.VMEM((GROUP, DD), jnp.float32),
        pltpu.VMEM((GROUP, DD), jnp.float32),
        pltpu.VMEM((GROUP, DD), jnp.float32),
        pltpu.VMEM((GROUP, DD), jnp.float32),
        pltpu.VMEM((4 * DD,), jnp.float32),
        pltpu.VMEM((LL,), jnp.float32),
        pltpu.VMEM((GROUP,), jnp.float32),
        pltpu.SemaphoreType.DMA,
    ]

    @functools.partial(pl.kernel, out_type=out_type, mesh=mesh,
                       scratch_types=scratch, compiler_params=_SC_PARAMS)
    def k(van_hbm, h_hbm, a1_hbm, a2_hbm, wc_hbm, bc_hbm, out_hbm,
          a1_v, a2_v, vb1, vb2, hb1, hb2, wc_v, bc_v, lbuf, sem):
        cid = lax.axis_index("c")
        sid = lax.axis_index("s")
        wid = sid * NCORE + cid
        pltpu.sync_copy(a1_hbm.at[wid], a1_v)
        pltpu.sync_copy(a2_hbm.at[wid], a2_v)
        pltpu.sync_copy(wc_hbm, wc_v)
        pltpu.sync_copy(bc_hbm, bc_v)
        iot = lax.iota(jnp.int32, LL)

        def gbody(g, carry):
            c1 = pltpu.async_copy(van_hbm.at[a1_v.at[g]], vb1, sem)
            c2 = pltpu.async_copy(van_hbm.at[a2_v.at[g]], vb2, sem)
            c3 = pltpu.async_copy(h_hbm.at[a1_v.at[g]], hb1, sem)
            c4 = pltpu.async_copy(h_hbm.at[a2_v.at[g]], hb2, sem)
            c1.wait()
            c2.wait()
            c3.wait()
            c4.wait()

            def sbody(k4, carry2):
                rowi = iot + k4 * LL
                acc = jnp.zeros((LL,), jnp.float32)
                for blk in range(DD // LL):
                    w0 = wc_v[pl.ds(blk * LL, LL)]
                    w1 = wc_v[pl.ds(DD + blk * LL, LL)]
                    w2 = wc_v[pl.ds(2 * DD + blk * LL, LL)]
                    w3 = wc_v[pl.ds(3 * DD + blk * LL, LL)]
                    for j in range(LL):
                        c = blk * LL + j
                        colv = jnp.full((LL,), c, jnp.int32)
                        acc = acc + plsc.load_gather(vb1, [rowi, colv]) * w0[j]
                        acc = acc + plsc.load_gather(vb2, [rowi, colv]) * w1[j]
                        acc = acc + plsc.load_gather(hb1, [rowi, colv]) * w2[j]
                        acc = acc + plsc.load_gather(hb2, [rowi, colv]) * w3[j]
                acc = acc + bc_v[pl.ds(0, LL)][0]
                lbuf[pl.ds(k4 * LL, LL)] = acc
                return carry2

            lax.fori_loop(0, GROUP // LL, sbody, 0)
            pltpu.sync_copy(
                lbuf, out_hbm.at[pl.ds(wid * (BGROUPS * GROUP) + g * GROUP,
                                       GROUP)])
            return carry

        lax.fori_loop(0, BGROUPS, gbody, 0)

    return k


_EDGE8 = _edge_pass(8)
_EDGE1 = _edge_pass(1)
_CLS = _cls_kernel()


def kernel(gnn_x, vanilla, edge_index, article1_idx, article2_idx,
           Wl1, Wr1, att1, b1, Wl2, Wr2, att2, b2, Wc, bc):
    loops = jnp.arange(NN, dtype=edge_index.dtype)
    src = jnp.concatenate([edge_index[0], loops])
    dst = jnp.concatenate([edge_index[1], loops])
    pad = EPAD - src.shape[0]
    padv = jnp.full((pad,), NN, dtype=jnp.int32)
    srcp = jnp.concatenate([src, padv]).reshape(NWORK, NGROUPS, GROUP)
    dstp = jnp.concatenate([dst, padv]).reshape(NWORK, NGROUPS, GROUP)
    xp = jnp.pad(_f32(gnn_x), ((0, NPAD - NN), (0, 0)))

    xl1, xr1 = _mm2(xp, _f32(Wl1), _f32(Wr1))
    u1, d1 = _EDGE8(xl1, xr1, srcp, dstp, _f32(att1).reshape(-1))
    h1, xl2, xr2 = _combine1(u1, d1, _f32(b1).reshape(1, DD),
                             _f32(Wl2), _f32(Wr2))

    u2, d2 = _EDGE1(xl2, xr2, srcp, dstp, _f32(att2).reshape(-1))
    h2 = _combine2(u2, d2, _f32(b2).reshape(1, DD))

    a1p = article1_idx.reshape(NWORK, BGROUPS, GROUP)
    a2p = article2_idx.reshape(NWORK, BGROUPS, GROUP)
    logits = _CLS(_f32(vanilla), h2, a1p, a2p,
                  _f32(Wc).reshape(-1), jnp.pad(_f32(bc), (0, LL - 1)))
    return logits.reshape(BB, 1)
